# baseline (speedup 1.0000x reference)
import functools

import jax
import jax.numpy as jnp
from jax import lax
from jax.experimental import pallas as pl
from jax.experimental.pallas import tpu as pltpu
from jax.experimental.pallas import tpu_sc as plsc

_EPS = 1e-12
_NW = 32
_CHUNK = 128


_NBUF = 5


def _sc_gather(word_emb, ids_flat, chunk):
    n = ids_flat.shape[0]
    hid = word_emb.shape[1]
    per_w = n // _NW
    steps = per_w // chunk
    nb = min(_NBUF, steps)
    mesh = plsc.VectorSubcoreMesh(core_axis_name="c", subcore_axis_name="s")

    scratch = [pltpu.VMEM((per_w,), jnp.int32)]
    scratch += [pltpu.VMEM((chunk, hid), jnp.float32) for _ in range(nb)]
    scratch += [pltpu.SemaphoreType.DMA for _ in range(2 * nb)]

    @functools.partial(
        pl.kernel,
        mesh=mesh,
        out_type=jax.ShapeDtypeStruct((n, hid), jnp.float32),
        scratch_types=scratch,
    )
    def k(tab_hbm, idx_hbm, out_hbm, idx_v, *rest):
        bufs = rest[:nb]
        gsems = rest[nb:2 * nb]
        wsems = rest[2 * nb:]
        wid = lax.axis_index("s") * 2 + lax.axis_index("c")
        base = pl.multiple_of(wid * per_w, 8)
        pltpu.sync_copy(idx_hbm.at[pl.ds(base, per_w)], idx_v)

        def gather(g):
            b = g % nb
            return pltpu.async_copy(
                tab_hbm.at[idx_v.at[pl.ds(g * chunk, chunk)]], bufs[b],
                gsems[b])

        def write(g):
            b = g % nb
            off = pl.multiple_of(base + g * chunk, 8)
            return pltpu.async_copy(bufs[b], out_hbm.at[pl.ds(off, chunk)],
                                    wsems[b])

        lead = max(nb - 2, 1)
        ghandles, whandles = {}, {}
        for g in range(min(lead, steps)):
            ghandles[g] = gather(g)
        for g in range(steps):
            ghandles.pop(g).wait()
            whandles[g] = write(g)
            ng = g + lead
            if ng < steps:
                prev_w = ng - nb
                if prev_w >= 0:
                    whandles.pop(prev_w).wait()
                ghandles[ng] = gather(ng)
        for g in sorted(whandles):
            whandles.pop(g).wait()

    return k(word_emb, ids_flat)


def _tc_ln(gathered, pos_tiled, gamma, beta, partial, split, total_rows, hid,
           rblk):
    rows = gathered.shape[0]
    nblk = rows // rblk
    nb2 = nblk // 2

    def norm(x, g, b):
        inv_h = 1.0 / hid
        mean = jnp.sum(x, axis=-1, keepdims=True) * inv_h
        sq = jnp.sum(x * x, axis=-1, keepdims=True) * inv_h
        var = sq - mean * mean
        r = lax.rsqrt(var + _EPS)
        return (x - mean) * (r * g) + b

    def body(x1_ref, x2_ref, pos_ref, g_ref, b_ref, *part_and_out):
        o_ref = part_and_out[-1]
        g = g_ref[...][0]
        b = b_ref[...][0]
        o_ref[:rblk] = norm(x1_ref[...] + pos_ref[...], g, b)
        o_ref[rblk:] = norm(x2_ref[...] + pos_ref[...], g, b)

    in_specs = [
        pl.BlockSpec((rblk, hid), lambda i: (2 * i, 0)),
        pl.BlockSpec((rblk, hid), lambda i: (2 * i + 1, 0)),
        pl.BlockSpec((rblk, hid), lambda i: (0, 0)),
        pl.BlockSpec((1, hid), lambda i: (0, 0)),
        pl.BlockSpec((1, hid), lambda i: (0, 0)),
    ]
    operands = [gathered, gathered, pos_tiled, gamma.reshape(1, hid),
                beta.reshape(1, hid)]
    aliases = {}
    if partial is not None:
        in_specs.append(pl.BlockSpec(memory_space=pl.ANY))
        operands.append(partial)
        aliases = {5: 0}

    return pl.pallas_call(
        body,
        grid=(nb2,),
        in_specs=in_specs,
        out_specs=pl.BlockSpec((2 * rblk, hid),
                               lambda i, s=split, n=nb2: (s * n + i, 0)),
        out_shape=jax.ShapeDtypeStruct((total_rows, hid), jnp.float32),
        input_output_aliases=aliases,
    )(*operands)


def kernel(input_ids, word_emb, pos_emb, ln_gamma, ln_beta):
    b, s = input_ids.shape
    hid = word_emb.shape[1]
    nsplit = 8
    ids = input_ids.reshape(nsplit, -1)
    rows = ids.shape[1]
    total_rows = b * s
    rblk = 8 * s
    pos_tiled = jnp.tile(pos_emb[:s], (8, 1))
    chunk = 160
    out = None
    for k in range(nsplit):
        gathered = _sc_gather(word_emb, ids[k], chunk)
        out = _tc_ln(gathered, pos_tiled, ln_gamma, ln_beta, out, k,
                     total_rows, hid, rblk)
    return out.reshape(b, s, hid)


# speedup vs baseline: 1.0019x; 1.0019x over previous
import functools

import jax
import jax.numpy as jnp
from jax import lax
from jax.experimental import pallas as pl
from jax.experimental.pallas import tpu as pltpu
from jax.experimental.pallas import tpu_sc as plsc

_EPS = 1e-12
_NW = 32
_CHUNK = 128


_NBUF = 5


def _sc_gather(word_emb, ids_flat, chunk):
    n = ids_flat.shape[0]
    hid = word_emb.shape[1]
    per_w = n // _NW
    steps = per_w // chunk
    nb = min(_NBUF, steps)
    mesh = plsc.VectorSubcoreMesh(core_axis_name="c", subcore_axis_name="s")

    scratch = [pltpu.VMEM((per_w,), jnp.int32)]
    scratch += [pltpu.VMEM((chunk, hid), jnp.float32) for _ in range(nb)]
    scratch += [pltpu.SemaphoreType.DMA for _ in range(2 * nb)]

    @functools.partial(
        pl.kernel,
        mesh=mesh,
        out_type=jax.ShapeDtypeStruct((n, hid), jnp.float32),
        scratch_types=scratch,
    )
    def k(tab_hbm, idx_hbm, out_hbm, idx_v, *rest):
        bufs = rest[:nb]
        gsems = rest[nb:2 * nb]
        wsems = rest[2 * nb:]
        wid = lax.axis_index("s") * 2 + lax.axis_index("c")
        base = pl.multiple_of(wid * per_w, 8)
        pltpu.sync_copy(idx_hbm.at[pl.ds(base, per_w)], idx_v)

        def gather(g):
            b = g % nb
            return pltpu.async_copy(
                tab_hbm.at[idx_v.at[pl.ds(g * chunk, chunk)]], bufs[b],
                gsems[b])

        def write(g):
            b = g % nb
            off = pl.multiple_of(base + g * chunk, 8)
            return pltpu.async_copy(bufs[b], out_hbm.at[pl.ds(off, chunk)],
                                    wsems[b])

        lead = max(nb - 2, 1)
        ghandles, whandles = {}, {}
        for g in range(min(lead, steps)):
            ghandles[g] = gather(g)
        for g in range(steps):
            ghandles.pop(g).wait()
            whandles[g] = write(g)
            ng = g + lead
            if ng < steps:
                prev_w = ng - nb
                if prev_w >= 0:
                    whandles.pop(prev_w).wait()
                ghandles[ng] = gather(ng)
        for g in sorted(whandles):
            whandles.pop(g).wait()

    return k(word_emb, ids_flat)


def _tc_ln(gathered, pos_tiled, gamma, beta, partial, split, total_rows, hid,
           rblk):
    rows = gathered.shape[0]
    nblk = rows // rblk
    nb2 = nblk // 2

    def norm(x, g, b):
        inv_h = 1.0 / hid
        mean = jnp.sum(x, axis=-1, keepdims=True) * inv_h
        sq = jnp.sum(x * x, axis=-1, keepdims=True) * inv_h
        var = sq - mean * mean
        r = lax.rsqrt(var + _EPS)
        return (x - mean) * (r * g) + b

    def body(x1_ref, x2_ref, pos_ref, g_ref, b_ref, *part_and_out):
        o_ref = part_and_out[-1]
        g = g_ref[...][0]
        b = b_ref[...][0]
        o_ref[:rblk] = norm(x1_ref[...] + pos_ref[...], g, b)
        o_ref[rblk:] = norm(x2_ref[...] + pos_ref[...], g, b)

    in_specs = [
        pl.BlockSpec((rblk, hid), lambda i: (2 * i, 0)),
        pl.BlockSpec((rblk, hid), lambda i: (2 * i + 1, 0)),
        pl.BlockSpec((rblk, hid), lambda i: (0, 0)),
        pl.BlockSpec((1, hid), lambda i: (0, 0)),
        pl.BlockSpec((1, hid), lambda i: (0, 0)),
    ]
    operands = [gathered, gathered, pos_tiled, gamma.reshape(1, hid),
                beta.reshape(1, hid)]
    aliases = {}
    if partial is not None:
        in_specs.append(pl.BlockSpec(memory_space=pl.ANY))
        operands.append(partial)
        aliases = {5: 0}

    return pl.pallas_call(
        body,
        grid=(nb2,),
        in_specs=in_specs,
        out_specs=pl.BlockSpec((2 * rblk, hid),
                               lambda i, s=split, n=nb2: (s * n + i, 0)),
        out_shape=jax.ShapeDtypeStruct((total_rows, hid), jnp.float32),
        input_output_aliases=aliases,
    )(*operands)


def kernel(input_ids, word_emb, pos_emb, ln_gamma, ln_beta):
    b, s = input_ids.shape
    hid = word_emb.shape[1]
    nsplit = 8
    ids = input_ids.reshape(nsplit, -1)
    rows = ids.shape[1]
    total_rows = b * s
    rblk = 16 * s
    pos_tiled = jnp.tile(pos_emb[:s], (16, 1))
    chunk = 200
    out = None
    for k in range(nsplit):
        gathered = _sc_gather(word_emb, ids[k], chunk)
        out = _tc_ln(gathered, pos_tiled, ln_gamma, ln_beta, out, k,
                     total_rows, hid, rblk)
    return out.reshape(b, s, hid)


# speedup vs baseline: 1.0078x; 1.0060x over previous
import functools

import jax
import jax.numpy as jnp
from jax import lax
from jax.experimental import pallas as pl
from jax.experimental.pallas import tpu as pltpu
from jax.experimental.pallas import tpu_sc as plsc

_EPS = 1e-12
_NW = 32
_CHUNK = 128


_NBUF = 5


def _sc_gather(word_emb, ids_flat, chunk):
    n = ids_flat.shape[0]
    hid = word_emb.shape[1]
    per_w = n // _NW
    steps = per_w // chunk
    nb = min(_NBUF, steps)
    mesh = plsc.VectorSubcoreMesh(core_axis_name="c", subcore_axis_name="s")

    scratch = [pltpu.VMEM((per_w,), jnp.int32)]
    scratch += [pltpu.VMEM((chunk, hid), jnp.float32) for _ in range(nb)]
    scratch += [pltpu.SemaphoreType.DMA for _ in range(2 * nb)]

    @functools.partial(
        pl.kernel,
        mesh=mesh,
        out_type=jax.ShapeDtypeStruct((n, hid), jnp.float32),
        scratch_types=scratch,
    )
    def k(tab_hbm, idx_hbm, out_hbm, idx_v, *rest):
        bufs = rest[:nb]
        gsems = rest[nb:2 * nb]
        wsems = rest[2 * nb:]
        wid = lax.axis_index("s") * 2 + lax.axis_index("c")
        base = pl.multiple_of(wid * per_w, 8)
        pltpu.sync_copy(idx_hbm.at[pl.ds(base, per_w)], idx_v)

        def gather(g):
            b = g % nb
            return pltpu.async_copy(
                tab_hbm.at[idx_v.at[pl.ds(g * chunk, chunk)]], bufs[b],
                gsems[b])

        def write(g):
            b = g % nb
            off = pl.multiple_of(base + g * chunk, 8)
            return pltpu.async_copy(bufs[b], out_hbm.at[pl.ds(off, chunk)],
                                    wsems[b])

        lead = max(nb - 2, 1)
        ghandles, whandles = {}, {}
        for g in range(min(lead, steps)):
            ghandles[g] = gather(g)
        for g in range(steps):
            ghandles.pop(g).wait()
            whandles[g] = write(g)
            ng = g + lead
            if ng < steps:
                prev_w = ng - nb
                if prev_w >= 0:
                    whandles.pop(prev_w).wait()
                ghandles[ng] = gather(ng)
        for g in sorted(whandles):
            whandles.pop(g).wait()

    return k(word_emb, ids_flat)


def _tc_ln(gathered, pos_tiled, gamma, beta, partial, split, total_rows, hid,
           rblk):
    rows = gathered.shape[0]
    nblk = rows // rblk
    nb2 = nblk // 2

    def norm(x, g, b):
        inv_h = 1.0 / hid
        mean = jnp.sum(x, axis=-1, keepdims=True) * inv_h
        sq = jnp.sum(x * x, axis=-1, keepdims=True) * inv_h
        var = sq - mean * mean
        r = lax.rsqrt(var + _EPS)
        return (x - mean) * (r * g) + b

    def body(x1_ref, x2_ref, pos_ref, g_ref, b_ref, *part_and_out):
        o_ref = part_and_out[-1]
        g = g_ref[...][0]
        b = b_ref[...][0]
        o_ref[:rblk] = norm(x1_ref[...] + pos_ref[...], g, b)
        o_ref[rblk:] = norm(x2_ref[...] + pos_ref[...], g, b)

    in_specs = [
        pl.BlockSpec((rblk, hid), lambda i: (2 * i, 0)),
        pl.BlockSpec((rblk, hid), lambda i: (2 * i + 1, 0)),
        pl.BlockSpec((rblk, hid), lambda i: (0, 0)),
        pl.BlockSpec((1, hid), lambda i: (0, 0)),
        pl.BlockSpec((1, hid), lambda i: (0, 0)),
    ]
    operands = [gathered, gathered, pos_tiled, gamma.reshape(1, hid),
                beta.reshape(1, hid)]
    aliases = {}
    if partial is not None:
        in_specs.append(pl.BlockSpec(memory_space=pl.ANY))
        operands.append(partial)
        aliases = {5: 0}

    return pl.pallas_call(
        body,
        grid=(nb2,),
        in_specs=in_specs,
        out_specs=pl.BlockSpec((2 * rblk, hid),
                               lambda i, s=split, n=nb2: (s * n + i, 0)),
        out_shape=jax.ShapeDtypeStruct((total_rows, hid), jnp.float32),
        input_output_aliases=aliases,
    )(*operands)


def kernel(input_ids, word_emb, pos_emb, ln_gamma, ln_beta):
    b, s = input_ids.shape
    hid = word_emb.shape[1]
    nsplit = 8
    ids = input_ids.reshape(nsplit, -1)
    rows = ids.shape[1]
    total_rows = b * s
    rblk = 16 * s
    pos_tiled = jnp.tile(pos_emb[:s], (16, 1))
    chunk = 160
    out = None
    for k in range(nsplit):
        gathered = _sc_gather(word_emb, ids[k], chunk)
        out = _tc_ln(gathered, pos_tiled, ln_gamma, ln_beta, out, k,
                     total_rows, hid, rblk)
    return out.reshape(b, s, hid)
